# Initial kernel scaffold; baseline (speedup 1.0000x reference)
import jax
import jax.numpy as jnp
from jax import lax
from jax.experimental import pallas as pl
from jax.experimental.pallas import tpu as pltpu
from jax.experimental.pallas import tpu_sc as plsc

B = 16384
NCAT = 26
NNUM = 13
ND = 32
NF = NCAT + NNUM

NC = 2
NS = 16
NW = NC * NS
BPW = B // NW
R = 32
NCH = BPW // R
G = 104
NG = R * NCAT // G


def _body(xcat_hbm, xnum_hbm, tbl_hbm, catb_hbm, numw_hbm, numb_hbm, out_hbm,
          idx_v, rows_v, out_v, xnum_v, catb_v, numw_v, numb_v, gsem):
    cid = lax.axis_index("c")
    sid = lax.axis_index("s")
    wid = sid * NC + cid
    base = wid * BPW

    pltpu.sync_copy(catb_hbm, catb_v)
    pltpu.sync_copy(numw_hbm, numw_v)
    pltpu.sync_copy(numb_hbm, numb_v)

    @pl.loop(0, NCH)
    def _chunk(c):
        r0 = base + c * R
        irow = r0 * NCAT // G
        pltpu.sync_copy(xcat_hbm.at[pl.ds(irow, NG)], idx_v)
        pltpu.sync_copy(xnum_hbm.at[pl.ds(r0, R)], xnum_v)
        descs = [
            pltpu.async_copy(tbl_hbm.at[idx_v.at[j]],
                             rows_v.at[pl.ds(j * G, G)], gsem)
            for j in range(NG)
        ]
        for d in descs:
            d.wait()

        @pl.loop(0, R)
        def _row(r):
            rb = r * NCAT
            for f in range(NCAT):
                for d0 in range(0, ND, 16):
                    out_v[r, f, pl.ds(d0, 16)] = (
                        rows_v[rb + f, pl.ds(d0, 16)] + catb_v[f, pl.ds(d0, 16)])
            for f in range(NNUM):
                s = xnum_v[r, f]
                for d0 in range(0, ND, 16):
                    out_v[r, NCAT + f, pl.ds(d0, 16)] = (
                        numw_v[f, pl.ds(d0, 16)] * s + numb_v[f, pl.ds(d0, 16)])

        pltpu.sync_copy(out_v, out_hbm.at[pl.ds(r0, R)])


def kernel(x_cat, x_num, emb_table, cat_bias, num_weight, num_bias):
    xcat_g = x_cat.astype(jnp.int32).reshape(B * NCAT // G, G)
    mesh = plsc.VectorSubcoreMesh(
        core_axis_name="c", subcore_axis_name="s",
        num_cores=NC, num_subcores=NS)
    out = pl.kernel(
        _body,
        out_type=jax.ShapeDtypeStruct((B, NF, ND), jnp.float32),
        mesh=mesh,
        scratch_types=[
            pltpu.VMEM((NG, G), jnp.int32),
            pltpu.VMEM((R * NCAT, ND), jnp.float32),
            pltpu.VMEM((R, NF, ND), jnp.float32),
            pltpu.VMEM((R, NNUM), jnp.float32),
            pltpu.VMEM((NCAT, ND), jnp.float32),
            pltpu.VMEM((NNUM, ND), jnp.float32),
            pltpu.VMEM((NNUM, ND), jnp.float32),
            pltpu.SemaphoreType.DMA,
        ],
    )(xcat_g, x_num, emb_table, cat_bias, num_weight, num_bias)
    return out.reshape(B, NF * ND)


# speedup vs baseline: 3.1888x; 3.1888x over previous
import jax
import jax.numpy as jnp
from jax import lax
from jax.experimental import pallas as pl
from jax.experimental.pallas import tpu as pltpu
from jax.experimental.pallas import tpu_sc as plsc

B = 16384
NCAT = 26
NNUM = 13
ND = 32
NF = NCAT + NNUM

NC = 2
NS = 16
NW = NC * NS
BPW = B // NW
R = 32
NCH = BPW // R
G = 104
NG = R * NCAT // G


def _body(xcat_hbm, xnum_hbm, tbl_hbm, catb_hbm, numw_hbm, numb_hbm, out_hbm,
          idx_v, rows_v, out_v, xnum_v, catb_v, numw_v, numb_v, gsem):
    cid = lax.axis_index("c")
    sid = lax.axis_index("s")
    wid = sid * NC + cid
    base = wid * BPW

    pltpu.sync_copy(catb_hbm, catb_v)
    pltpu.sync_copy(numw_hbm, numw_v)
    pltpu.sync_copy(numb_hbm, numb_v)

    @pl.loop(0, NCH)
    def _chunk(c):
        r0 = pl.multiple_of(base + c * R, R)
        irow = pl.multiple_of((base + c * R) * NCAT // G, 8)
        pltpu.sync_copy(xcat_hbm.at[pl.ds(irow, NG)], idx_v)
        pltpu.sync_copy(xnum_hbm.at[pl.ds(r0, R)], xnum_v)
        descs = [
            pltpu.async_copy(tbl_hbm.at[idx_v.at[j]],
                             rows_v.at[pl.ds(j * G, G)], gsem)
            for j in range(NG)
        ]
        for d in descs:
            d.wait()

        @pl.loop(0, R)
        def _row(r):
            rb = r * NCAT
            for f in range(NCAT):
                for d0 in range(0, ND, 16):
                    out_v[r, f, pl.ds(d0, 16)] = (
                        rows_v[rb + f, pl.ds(d0, 16)] + catb_v[f, pl.ds(d0, 16)])
            xv = xnum_v[r, :]
            for f in range(NNUM):
                s = xv[f]
                for d0 in range(0, ND, 16):
                    out_v[r, NCAT + f, pl.ds(d0, 16)] = (
                        numw_v[f, pl.ds(d0, 16)] * s + numb_v[f, pl.ds(d0, 16)])

        pltpu.sync_copy(out_v, out_hbm.at[pl.ds(r0, R)])


def kernel(x_cat, x_num, emb_table, cat_bias, num_weight, num_bias):
    xcat_g = x_cat.astype(jnp.int32).reshape(B * NCAT // G, G)
    xnum_p = jnp.pad(x_num, ((0, 0), (0, 16 - NNUM)))
    mesh = plsc.VectorSubcoreMesh(
        core_axis_name="c", subcore_axis_name="s",
        num_cores=NC, num_subcores=NS)
    out = pl.kernel(
        _body,
        out_type=jax.ShapeDtypeStruct((B, NF, ND), jnp.float32),
        mesh=mesh,
        compiler_params=pltpu.CompilerParams(use_tc_tiling_on_sc=False),
        scratch_types=[
            pltpu.VMEM((NG, G), jnp.int32),
            pltpu.VMEM((R * NCAT, ND), jnp.float32),
            pltpu.VMEM((R, NF, ND), jnp.float32),
            pltpu.VMEM((R, 16), jnp.float32),
            pltpu.VMEM((NCAT, ND), jnp.float32),
            pltpu.VMEM((NNUM, ND), jnp.float32),
            pltpu.VMEM((NNUM, ND), jnp.float32),
            pltpu.SemaphoreType.DMA,
        ],
    )(xcat_g, xnum_p, emb_table, cat_bias, num_weight, num_bias)
    return out.reshape(B, NF * ND)


# speedup vs baseline: 3.5164x; 1.1027x over previous
import jax
import jax.numpy as jnp
from jax import lax
from jax.experimental import pallas as pl
from jax.experimental.pallas import tpu as pltpu
from jax.experimental.pallas import tpu_sc as plsc

B = 16384
NCAT = 26
NNUM = 13
ND = 32
NF = NCAT + NNUM

NC = 2
NS = 16
NW = NC * NS
BPW = B // NW
R = 16
NCH = BPW // R
G = 104
NGPC = R * NCAT // G
NGW = BPW * NCAT // G


def _body(xcat_hbm, xnum_hbm, tbl_hbm, catb_hbm, numw_hbm, numb_hbm, out_hbm,
          idx_all, xnum_all, rows0, rows1, out0, out1, catb_v, numw_v, numb_v,
          gsem0, gsem1, osem0, osem1):
    rows = [rows0, rows1]
    outs = [out0, out1]
    gsems = [gsem0, gsem1]
    osems = [osem0, osem1]

    cid = lax.axis_index("c")
    sid = lax.axis_index("s")
    wid = sid * NC + cid
    base = wid * BPW

    pltpu.sync_copy(catb_hbm, catb_v)
    pltpu.sync_copy(numw_hbm, numw_v)
    pltpu.sync_copy(numb_hbm, numb_v)
    pltpu.sync_copy(xcat_hbm.at[pl.ds(pl.multiple_of(wid * NGW, 8), NGW)],
                    idx_all)
    pltpu.sync_copy(xnum_hbm.at[pl.ds(pl.multiple_of(base, 8), BPW)], xnum_all)

    def issue(cc, p):
        for j in range(NGPC):
            pltpu.async_copy(tbl_hbm.at[idx_all.at[cc * NGPC + j]],
                             rows[p].at[pl.ds(j * G, G)], gsems[p])

    def drain(p):
        for j in range(NGPC):
            pltpu.make_async_copy(tbl_hbm.at[idx_all.at[0]],
                                  rows[p].at[pl.ds(j * G, G)], gsems[p]).wait()

    def outwait(cc, p):
        pltpu.make_async_copy(
            outs[p], out_hbm.at[pl.ds(pl.multiple_of(base, 8), R)],
            osems[p]).wait()

    def compute(cc, p):
        rows_v = rows[p]
        out_v = outs[p]

        @pl.loop(0, R)
        def _row(r):
            rb = r * NCAT
            for f in range(NCAT):
                for d0 in range(0, ND, 16):
                    out_v[r, f, pl.ds(d0, 16)] = (
                        rows_v[rb + f, pl.ds(d0, 16)]
                        + catb_v[f, pl.ds(d0, 16)])
            xv = xnum_all[cc * R + r, :]
            for f in range(NNUM):
                s = xv[f]
                for d0 in range(0, ND, 16):
                    out_v[r, NCAT + f, pl.ds(d0, 16)] = (
                        numw_v[f, pl.ds(d0, 16)] * s + numb_v[f, pl.ds(d0, 16)])

    def outdma(cc, p):
        r0 = pl.multiple_of(base + cc * R, R)
        pltpu.async_copy(outs[p], out_hbm.at[pl.ds(r0, R)], osems[p])

    issue(0, 0)
    issue(1, 1)

    @pl.loop(0, NCH, step=2)
    def _chunks(c):
        for b in range(2):
            cc = c + b
            p = b
            drain(p)

            @pl.when(cc >= 2)
            def _():
                outwait(cc, p)

            compute(cc, p)
            outdma(cc, p)

            @pl.when(cc + 2 < NCH)
            def _():
                issue(cc + 2, p)

    outwait(NCH - 2, 0)
    outwait(NCH - 1, 1)


def kernel(x_cat, x_num, emb_table, cat_bias, num_weight, num_bias):
    xcat_g = x_cat.astype(jnp.int32).reshape(B * NCAT // G, G)
    xnum_p = jnp.pad(x_num, ((0, 0), (0, 16 - NNUM)))
    mesh = plsc.VectorSubcoreMesh(
        core_axis_name="c", subcore_axis_name="s",
        num_cores=NC, num_subcores=NS)
    out = pl.kernel(
        _body,
        out_type=jax.ShapeDtypeStruct((B, NF, ND), jnp.float32),
        mesh=mesh,
        compiler_params=pltpu.CompilerParams(use_tc_tiling_on_sc=False),
        scratch_types=[
            pltpu.VMEM((NGW, G), jnp.int32),
            pltpu.VMEM((BPW, 16), jnp.float32),
            pltpu.VMEM((R * NCAT, ND), jnp.float32),
            pltpu.VMEM((R * NCAT, ND), jnp.float32),
            pltpu.VMEM((R, NF, ND), jnp.float32),
            pltpu.VMEM((R, NF, ND), jnp.float32),
            pltpu.VMEM((NCAT, ND), jnp.float32),
            pltpu.VMEM((NNUM, ND), jnp.float32),
            pltpu.VMEM((NNUM, ND), jnp.float32),
            pltpu.SemaphoreType.DMA,
            pltpu.SemaphoreType.DMA,
            pltpu.SemaphoreType.DMA,
            pltpu.SemaphoreType.DMA,
        ],
    )(xcat_g, xnum_p, emb_table, cat_bias, num_weight, num_bias)
    return out.reshape(B, NF * ND)
